# baseline (speedup 1.0000x reference)
import functools

import jax
import jax.numpy as jnp
from jax import lax
from jax.experimental import pallas as pl
from jax.experimental.pallas import tpu as pltpu
from jax.experimental.pallas import tpu_sc as plsc

N = 10000
D = 128
E = 320000
NC = 2
NS = 16
NW = NC * NS
E_W = E // NW
N_PAD = 10240
ROWS_W = N_PAD // NS
E_WP = 10240
CHS = 80
NCHS = E_WP // CHS
CHG = 160
BLK_G = 4
E_BLK = BLK_G * CHG
BLK_S = E_BLK // CHS
NBLK = E_WP // E_BLK

_mesh = plsc.VectorSubcoreMesh(core_axis_name="c", subcore_axis_name="s")


@functools.partial(
    pl.kernel,
    out_type=jax.ShapeDtypeStruct((NC, 1, N_PAD), jnp.float32),
    mesh=_mesh,
    scratch_types=[
        pltpu.VMEM((NCHS, CHS), jnp.int32),
        pltpu.VMEM((CHS,), jnp.float32),
        pltpu.VMEM((ROWS_W,), jnp.float32),
        pltpu.VMEM_SHARED((N_PAD,), jnp.float32),
        pltpu.SemaphoreType.DMA,
    ],
)
def _deg_partials(dst_hbm, deg_out, didx, onesv, zb, acc, sem):
    c = lax.axis_index("c")
    s = lax.axis_index("s")
    wid = s * NC + c
    for i in range(CHS // 16):
        onesv[pl.ds(i * 16, 16)] = jnp.full((16,), 1.0, jnp.float32)
    for i in range(ROWS_W // 16):
        zb[pl.ds(i * 16, 16)] = jnp.zeros((16,), jnp.float32)
    pltpu.sync_copy(dst_hbm.at[wid], didx)
    pltpu.sync_copy(zb, acc.at[pl.ds(s * ROWS_W, ROWS_W)])
    plsc.subcore_barrier()

    def ebody(i, carry):
        pltpu.async_copy(onesv, acc.at[didx.at[i]], sem, add=True)
        return carry

    lax.fori_loop(0, NCHS, ebody, 0)

    def dbody(i, carry):
        pltpu.make_async_copy(onesv, acc.at[didx.at[0]], sem).wait()
        return carry

    lax.fori_loop(0, NCHS, dbody, 0)
    plsc.subcore_barrier()
    pltpu.sync_copy(acc.at[pl.ds(s * ROWS_W, ROWS_W)],
                    deg_out.at[c, 0, pl.ds(s * ROWS_W, ROWS_W)])


@functools.partial(
    pl.kernel,
    out_type=jax.ShapeDtypeStruct((NC, N_PAD, D), jnp.float32),
    mesh=_mesh,
    scratch_types=[
        pltpu.VMEM((E_BLK,), jnp.int32),
        pltpu.VMEM((BLK_S, CHS), jnp.int32),
        pltpu.VMEM((CHG, D), jnp.float32),
        pltpu.VMEM((CHG, D), jnp.float32),
        pltpu.VMEM_SHARED((N_PAD, D), jnp.float32),
        pltpu.SemaphoreType.DMA,
        pltpu.SemaphoreType.DMA,
    ],
)
def _agg_partials(hs_hbm, src_hbm, dst_hbm, out_hbm,
                  sidx, didx, rows0, rows1, acc, gsem0, gsem1):
    c = lax.axis_index("c")
    s = lax.axis_index("s")
    wid = s * NC + c
    rbase = s * ROWS_W
    rows = (rows0, rows1)
    gsem = (gsem0, gsem1)

    for i in range(32):
        for j in range(D // 16):
            rows1[i, pl.ds(j * 16, 16)] = jnp.zeros((16,), jnp.float32)

    def zbody(i, carry):
        pltpu.sync_copy(rows1.at[pl.ds(0, 32)], acc.at[pl.ds(rbase + i * 32, 32)])
        return carry

    lax.fori_loop(0, ROWS_W // 32, zbody, 0)
    plsc.subcore_barrier()

    def blk_body(blk, carry):
        pltpu.sync_copy(src_hbm.at[wid, 0, pl.ds(blk * E_BLK, E_BLK)], sidx)
        pltpu.sync_copy(dst_hbm.at[wid, pl.ds(blk * BLK_S, BLK_S)], didx)
        pltpu.async_copy(hs_hbm.at[sidx.at[pl.ds(0, CHG)]], rows0, gsem0)
        for k in range(1, BLK_G + 1):
            b = k % 2
            pb = (k - 1) % 2
            if k < BLK_G:
                pltpu.async_copy(hs_hbm.at[sidx.at[pl.ds(k * CHG, CHG)]],
                                 rows[b], gsem[b])
            pltpu.make_async_copy(hs_hbm.at[pl.ds(0, CHG)],
                                  rows[pb], gsem[pb]).wait()
            for j in range(CHG // CHS):
                pltpu.sync_copy(rows[pb].at[pl.ds(j * CHS, CHS)],
                                acc.at[didx.at[(k - 1) * (CHG // CHS) + j]],
                                add=True)
        return carry

    lax.fori_loop(0, NBLK, blk_body, 0)
    plsc.subcore_barrier()
    pltpu.sync_copy(acc.at[pl.ds(rbase, ROWS_W)],
                    out_hbm.at[c, pl.ds(rbase, ROWS_W)])


_BLK = 2000


def _mm2p_body(x_ref, wa_ref, wb_ref, d0_ref, d1_ref, oa_ref, ob_ref):
    x = x_ref[...]
    dinv = lax.rsqrt(d0_ref[...] + d1_ref[...] + 1.0)
    oa_ref[...] = jnp.dot(x, wa_ref[...], preferred_element_type=jnp.float32) * dinv
    ob_ref[...] = jnp.dot(x, wb_ref[...], preferred_element_type=jnp.float32)


def _combine_body(p_ref, hs_ref, d0_ref, d1_ref, b_ref, w_ref, o_ref):
    dinv = lax.rsqrt(d0_ref[...] + d1_ref[...] + 1.0)
    z = dinv * (p_ref[0] + p_ref[1] + hs_ref[...]) + b_ref[...]
    z = jnp.maximum(z, 0.0)
    o_ref[...] = jnp.dot(z, w_ref[...], preferred_element_type=jnp.float32) * dinv


def _final_body(p_ref, hs_ref, d0_ref, d1_ref, b_ref, w_ref, xc_ref,
                bc_ref, o_ref):
    dinv = lax.rsqrt(d0_ref[...] + d1_ref[...] + 1.0)
    z = dinv * (p_ref[0] + p_ref[1] + hs_ref[...]) + b_ref[...]
    z = jnp.maximum(z, 0.0)
    o_ref[...] = (jnp.dot(z, w_ref[...], preferred_element_type=jnp.float32)
                  + xc_ref[...] + bc_ref[...])


def _row_spec(w):
    return pl.BlockSpec((_BLK, w), lambda i: (i, 0))


def _full_spec(r, w):
    return pl.BlockSpec((r, w), lambda i: (0, 0))


_part_spec = pl.BlockSpec((2, _BLK, D), lambda i: (0, i, 0))

_G = N // _BLK

_mm2p = pl.pallas_call(
    _mm2p_body,
    grid=(_G,),
    in_specs=[_row_spec(D), _full_spec(D, D), _full_spec(D, 64),
              _row_spec(1), _row_spec(1)],
    out_specs=[_row_spec(D), _row_spec(64)],
    out_shape=[jax.ShapeDtypeStruct((N, D), jnp.float32),
               jax.ShapeDtypeStruct((N, 64), jnp.float32)],
)

_combine = pl.pallas_call(
    _combine_body,
    grid=(_G,),
    in_specs=[_part_spec, _row_spec(D), _row_spec(1),
              _row_spec(1), _full_spec(1, D), _full_spec(D, D)],
    out_specs=_row_spec(D),
    out_shape=jax.ShapeDtypeStruct((N, D), jnp.float32),
)

_final = pl.pallas_call(
    _final_body,
    grid=(_G,),
    in_specs=[_part_spec, _row_spec(D), _row_spec(1),
              _row_spec(1), _full_spec(1, D), _full_spec(D, 64),
              _row_spec(64), _full_spec(1, 64)],
    out_specs=_row_spec(64),
    out_shape=jax.ShapeDtypeStruct((N, 64), jnp.float32),
)


def _pad_edges(src, dst):
    npad = E_WP - E_W
    w = jnp.arange(NW, dtype=jnp.int32).reshape(NW, 1)
    k = jnp.arange(npad, dtype=jnp.int32).reshape(1, npad)
    pad_src = (k * 41 + w * 13) % N
    pad_dst = N + (k + w * 7) % npad
    srcp = jnp.concatenate([src.reshape(NW, E_W), pad_src], axis=1)
    dstp = jnp.concatenate([dst.reshape(NW, E_W), pad_dst], axis=1)
    return srcp.reshape(NW, 1, E_WP), dstp.reshape(NW, NCHS, CHS)


def kernel(x, edge_index, W1, b1, W2, b2, Wc, bc):
    src = edge_index[0].astype(jnp.int32)
    dst = edge_index[1].astype(jnp.int32)
    srcp, dstp = _pad_edges(src, dst)

    degp = _deg_partials(dstp)
    d0 = degp[0, 0, :N].reshape(N, 1)
    d1 = degp[1, 0, :N].reshape(N, 1)

    hs1, xc = _mm2p(x, W1, Wc[D:], d0, d1)
    p1 = _agg_partials(hs1, srcp, dstp)
    hs2 = _combine(p1, hs1, d0, d1,
                   b1.reshape(1, D), W2)
    p2 = _agg_partials(hs2, srcp, dstp)
    out = _final(p2, hs2, d0, d1,
                 b2.reshape(1, D), Wc[:D], xc,
                 bc.reshape(1, 64))
    return out


# speedup vs baseline: 1.1199x; 1.1199x over previous
import functools

import jax
import jax.numpy as jnp
from jax import lax
from jax.experimental import pallas as pl
from jax.experimental.pallas import tpu as pltpu
from jax.experimental.pallas import tpu_sc as plsc

N = 10000
D = 128
E = 320000
NC = 2
NS = 16
NW = NC * NS
E_W = E // NW
N_PAD = 10240
ROWS_W = N_PAD // NS
E_WP = 10240
CHS = 80
NCHS = E_WP // CHS
PAIR_BLK = 8
E_BLK = 2 * PAIR_BLK * CHS
NBLK = E_WP // E_BLK

_mesh = plsc.VectorSubcoreMesh(core_axis_name="c", subcore_axis_name="s")


@functools.partial(
    pl.kernel,
    out_type=jax.ShapeDtypeStruct((NC, 1, N_PAD), jnp.float32),
    mesh=_mesh,
    scratch_types=[
        pltpu.VMEM((NCHS, CHS), jnp.int32),
        pltpu.VMEM((CHS,), jnp.float32),
        pltpu.VMEM((ROWS_W,), jnp.float32),
        pltpu.VMEM_SHARED((N_PAD,), jnp.float32),
        pltpu.SemaphoreType.DMA,
    ],
)
def _deg_partials(dst_hbm, deg_out, didx, onesv, zb, acc, sem):
    c = lax.axis_index("c")
    s = lax.axis_index("s")
    wid = s * NC + c
    for i in range(CHS // 16):
        onesv[pl.ds(i * 16, 16)] = jnp.full((16,), 1.0, jnp.float32)
    for i in range(ROWS_W // 16):
        zb[pl.ds(i * 16, 16)] = jnp.zeros((16,), jnp.float32)
    pltpu.sync_copy(dst_hbm.at[wid], didx)
    pltpu.sync_copy(zb, acc.at[pl.ds(s * ROWS_W, ROWS_W)])
    plsc.subcore_barrier()

    def ebody(i, carry):
        pltpu.async_copy(onesv, acc.at[didx.at[i]], sem, add=True)
        return carry

    lax.fori_loop(0, NCHS, ebody, 0)

    def dbody(i, carry):
        pltpu.make_async_copy(onesv, acc.at[didx.at[0]], sem).wait()
        return carry

    lax.fori_loop(0, NCHS, dbody, 0)
    plsc.subcore_barrier()
    pltpu.sync_copy(acc.at[pl.ds(s * ROWS_W, ROWS_W)],
                    deg_out.at[c, 0, pl.ds(s * ROWS_W, ROWS_W)])


@functools.partial(
    pl.kernel,
    out_type=jax.ShapeDtypeStruct((NC, N_PAD, D), jnp.float32),
    mesh=_mesh,
    scratch_types=[
        pltpu.VMEM((E_BLK,), jnp.int32),
        pltpu.VMEM((2 * PAIR_BLK, CHS), jnp.int32),
        pltpu.VMEM((CHS, D), jnp.float32),
        pltpu.VMEM((CHS, D), jnp.float32),
        pltpu.VMEM((CHS, D), jnp.float32),
        pltpu.VMEM((CHS, D), jnp.float32),
        pltpu.VMEM_SHARED((N_PAD, D), jnp.float32),
        pltpu.SemaphoreType.DMA,
        pltpu.SemaphoreType.DMA,
        pltpu.SemaphoreType.DMA,
        pltpu.SemaphoreType.DMA,
        pltpu.SemaphoreType.DMA,
        pltpu.SemaphoreType.DMA,
    ],
)
def _agg_partials(hs_hbm, src_hbm, dst_hbm, out_hbm,
                  sidx, didx, a0, a1, b0, b1, acc,
                  ga0, ga1, gb0, gb1, ssem0, ssem1):
    c = lax.axis_index("c")
    s = lax.axis_index("s")
    wid = s * NC + c
    rbase = s * ROWS_W
    bufs = ((a0, a1), (b0, b1))
    gsems = ((ga0, ga1), (gb0, gb1))

    for i in range(32):
        for j in range(D // 16):
            a1[i, pl.ds(j * 16, 16)] = jnp.zeros((16,), jnp.float32)

    def zbody(i, carry):
        pltpu.sync_copy(a1.at[pl.ds(0, 32)], acc.at[pl.ds(rbase + i * 32, 32)])
        return carry

    lax.fori_loop(0, ROWS_W // 32, zbody, 0)
    plsc.subcore_barrier()

    def _gpair(p, par):
        for h in range(2):
            pltpu.async_copy(hs_hbm.at[sidx.at[pl.ds((2 * p + h) * CHS, CHS)]],
                             bufs[par][h], gsems[par][h])

    def _gwait(par):
        for h in range(2):
            pltpu.make_async_copy(hs_hbm.at[pl.ds(0, CHS)],
                                  bufs[par][h], gsems[par][h]).wait()

    def blk_body(blk, carry):
        pltpu.sync_copy(src_hbm.at[wid, 0, pl.ds(blk * E_BLK, E_BLK)], sidx)
        pltpu.sync_copy(dst_hbm.at[wid, pl.ds(blk * 2 * PAIR_BLK, 2 * PAIR_BLK)],
                        didx)
        _gpair(0, 0)
        for p in range(PAIR_BLK):
            par = p % 2
            if p + 1 < PAIR_BLK:
                _gpair(p + 1, 1 - par)
            _gwait(par)
            d0 = pltpu.async_copy(bufs[par][0], acc.at[didx.at[2 * p]],
                                  ssem0, add=True)
            d1 = pltpu.async_copy(bufs[par][1], acc.at[didx.at[2 * p + 1]],
                                  ssem1, add=True)
            d0.wait()
            d1.wait()
        return carry

    lax.fori_loop(0, NBLK, blk_body, 0)
    plsc.subcore_barrier()
    pltpu.sync_copy(acc.at[pl.ds(rbase, ROWS_W)],
                    out_hbm.at[c, pl.ds(rbase, ROWS_W)])


_BLK = 2000


def _mm2p_body(x_ref, wa_ref, wb_ref, d0_ref, d1_ref, oa_ref, ob_ref):
    x = x_ref[...]
    dinv = lax.rsqrt(d0_ref[...] + d1_ref[...] + 1.0)
    oa_ref[...] = jnp.dot(x, wa_ref[...], preferred_element_type=jnp.float32) * dinv
    ob_ref[...] = jnp.dot(x, wb_ref[...], preferred_element_type=jnp.float32)


def _combine_body(p_ref, hs_ref, d0_ref, d1_ref, b_ref, w_ref, o_ref):
    dinv = lax.rsqrt(d0_ref[...] + d1_ref[...] + 1.0)
    z = dinv * (p_ref[0] + p_ref[1] + hs_ref[...]) + b_ref[...]
    z = jnp.maximum(z, 0.0)
    o_ref[...] = jnp.dot(z, w_ref[...], preferred_element_type=jnp.float32) * dinv


def _final_body(p_ref, hs_ref, d0_ref, d1_ref, b_ref, w_ref, xc_ref,
                bc_ref, o_ref):
    dinv = lax.rsqrt(d0_ref[...] + d1_ref[...] + 1.0)
    z = dinv * (p_ref[0] + p_ref[1] + hs_ref[...]) + b_ref[...]
    z = jnp.maximum(z, 0.0)
    o_ref[...] = (jnp.dot(z, w_ref[...], preferred_element_type=jnp.float32)
                  + xc_ref[...] + bc_ref[...])


def _row_spec(w):
    return pl.BlockSpec((_BLK, w), lambda i: (i, 0))


def _full_spec(r, w):
    return pl.BlockSpec((r, w), lambda i: (0, 0))


_part_spec = pl.BlockSpec((2, _BLK, D), lambda i: (0, i, 0))

_G = N // _BLK

_mm2p = pl.pallas_call(
    _mm2p_body,
    grid=(_G,),
    in_specs=[_row_spec(D), _full_spec(D, D), _full_spec(D, 64),
              _row_spec(1), _row_spec(1)],
    out_specs=[_row_spec(D), _row_spec(64)],
    out_shape=[jax.ShapeDtypeStruct((N, D), jnp.float32),
               jax.ShapeDtypeStruct((N, 64), jnp.float32)],
)

_combine = pl.pallas_call(
    _combine_body,
    grid=(_G,),
    in_specs=[_part_spec, _row_spec(D), _row_spec(1),
              _row_spec(1), _full_spec(1, D), _full_spec(D, D)],
    out_specs=_row_spec(D),
    out_shape=jax.ShapeDtypeStruct((N, D), jnp.float32),
)

_final = pl.pallas_call(
    _final_body,
    grid=(_G,),
    in_specs=[_part_spec, _row_spec(D), _row_spec(1),
              _row_spec(1), _full_spec(1, D), _full_spec(D, 64),
              _row_spec(64), _full_spec(1, 64)],
    out_specs=_row_spec(64),
    out_shape=jax.ShapeDtypeStruct((N, 64), jnp.float32),
)


def _pad_edges(src, dst):
    npad = E_WP - E_W
    w = jnp.arange(NW, dtype=jnp.int32).reshape(NW, 1)
    k = jnp.arange(npad, dtype=jnp.int32).reshape(1, npad)
    pad_src = (k * 41 + w * 13) % N
    pad_dst = N + (k + w * 7) % npad
    srcp = jnp.concatenate([src.reshape(NW, E_W), pad_src], axis=1)
    dstp = jnp.concatenate([dst.reshape(NW, E_W), pad_dst], axis=1)
    return srcp.reshape(NW, 1, E_WP), dstp.reshape(NW, NCHS, CHS)


def kernel(x, edge_index, W1, b1, W2, b2, Wc, bc):
    src = edge_index[0].astype(jnp.int32)
    dst = edge_index[1].astype(jnp.int32)
    srcp, dstp = _pad_edges(src, dst)

    degp = _deg_partials(dstp)
    d0 = degp[0, 0, :N].reshape(N, 1)
    d1 = degp[1, 0, :N].reshape(N, 1)

    hs1, xc = _mm2p(x, W1, Wc[D:], d0, d1)
    p1 = _agg_partials(hs1, srcp, dstp)
    hs2 = _combine(p1, hs1, d0, d1,
                   b1.reshape(1, D), W2)
    p2 = _agg_partials(hs2, srcp, dstp)
    out = _final(p2, hs2, d0, d1,
                 b2.reshape(1, D), Wc[:D], xc,
                 bc.reshape(1, 64))
    return out


# speedup vs baseline: 1.1972x; 1.0690x over previous
import functools

import jax
import jax.numpy as jnp
from jax import lax
from jax.experimental import pallas as pl
from jax.experimental.pallas import tpu as pltpu
from jax.experimental.pallas import tpu_sc as plsc

N = 10000
D = 128
E = 320000
NC = 2
NS = 16
NW = NC * NS
E_W = E // NW
N_PAD = 10240
ROWS_W = N_PAD // NS
E_WP = 10240
CHP = 128
NCHP = E_WP // CHP
BLK_CH = 16
E_BLK = BLK_CH * CHP
NBLK = E_WP // E_BLK

_mesh = plsc.VectorSubcoreMesh(core_axis_name="c", subcore_axis_name="s")


@functools.partial(
    pl.kernel,
    out_type=jax.ShapeDtypeStruct((NC, 1, N_PAD), jnp.float32),
    mesh=_mesh,
    scratch_types=[
        pltpu.VMEM((NCHP, CHP), jnp.int32),
        pltpu.VMEM((CHP,), jnp.float32),
        pltpu.VMEM((ROWS_W,), jnp.float32),
        pltpu.VMEM_SHARED((N_PAD,), jnp.float32),
        pltpu.SemaphoreType.DMA,
    ],
)
def _deg_partials(dst_hbm, deg_out, didx, onesv, zb, acc, sem):
    c = lax.axis_index("c")
    s = lax.axis_index("s")
    wid = s * NC + c
    for i in range(CHP // 16):
        onesv[pl.ds(i * 16, 16)] = jnp.full((16,), 1.0, jnp.float32)
    for i in range(ROWS_W // 16):
        zb[pl.ds(i * 16, 16)] = jnp.zeros((16,), jnp.float32)
    pltpu.sync_copy(dst_hbm.at[wid], didx)
    pltpu.sync_copy(zb, acc.at[pl.ds(s * ROWS_W, ROWS_W)])
    plsc.subcore_barrier()

    def ebody(i, carry):
        pltpu.async_copy(onesv, acc.at[didx.at[i]], sem, add=True)
        return carry

    lax.fori_loop(0, NCHP, ebody, 0)

    def dbody(i, carry):
        pltpu.make_async_copy(onesv, acc.at[didx.at[0]], sem).wait()
        return carry

    lax.fori_loop(0, NCHP, dbody, 0)
    plsc.subcore_barrier()
    pltpu.sync_copy(acc.at[pl.ds(s * ROWS_W, ROWS_W)],
                    deg_out.at[c, 0, pl.ds(s * ROWS_W, ROWS_W)])


@functools.partial(
    pl.kernel,
    out_type=jax.ShapeDtypeStruct((NC, N_PAD, D), jnp.float32),
    mesh=_mesh,
    scratch_types=[
        pltpu.VMEM((BLK_CH, CHP), jnp.int32),
        pltpu.VMEM((BLK_CH, CHP), jnp.int32),
        pltpu.VMEM((CHP, D), jnp.float32),
        pltpu.VMEM((CHP, D), jnp.float32),
        pltpu.VMEM_SHARED((N_PAD, D), jnp.float32),
        pltpu.SemaphoreType.DMA,
        pltpu.SemaphoreType.DMA,
        pltpu.SemaphoreType.DMA,
        pltpu.SemaphoreType.DMA,
    ],
)
def _agg_partials(hs_hbm, src_hbm, dst_hbm, out_hbm,
                  sidx, didx, rows0, rows1, acc, gsem0, gsem1, ssem0, ssem1):
    c = lax.axis_index("c")
    s = lax.axis_index("s")
    wid = s * NC + c
    rbase = s * ROWS_W
    rows = (rows0, rows1)
    gsem = (gsem0, gsem1)
    ssem = (ssem0, ssem1)

    for i in range(32):
        for j in range(D // 16):
            rows1[i, pl.ds(j * 16, 16)] = jnp.zeros((16,), jnp.float32)

    def zbody(i, carry):
        pltpu.sync_copy(rows1.at[pl.ds(0, 32)], acc.at[pl.ds(rbase + i * 32, 32)])
        return carry

    lax.fori_loop(0, ROWS_W // 32, zbody, 0)
    plsc.subcore_barrier()

    def blk_body(blk, carry):
        pltpu.sync_copy(src_hbm.at[wid, pl.ds(blk * BLK_CH, BLK_CH)], sidx)
        pltpu.sync_copy(dst_hbm.at[wid, pl.ds(blk * BLK_CH, BLK_CH)], didx)
        descs = [None, None]
        for k in range(BLK_CH):
            b = k % 2
            if descs[b] is not None:
                descs[b].wait()
            pltpu.async_copy(hs_hbm.at[sidx.at[k]], rows[b], gsem[b])
            if k >= 1:
                pb = (k - 1) % 2
                pltpu.make_async_copy(hs_hbm.at[pl.ds(0, CHP)],
                                      rows[pb], gsem[pb]).wait()
                descs[pb] = pltpu.async_copy(rows[pb], acc.at[didx.at[k - 1]],
                                             ssem[pb], add=True)
        lb = (BLK_CH - 1) % 2
        pltpu.make_async_copy(hs_hbm.at[pl.ds(0, CHP)], rows[lb],
                              gsem[lb]).wait()
        dlast = pltpu.async_copy(rows[lb], acc.at[didx.at[BLK_CH - 1]],
                                 ssem[lb], add=True)
        descs[1 - lb].wait()
        dlast.wait()
        return carry

    lax.fori_loop(0, NBLK, blk_body, 0)
    plsc.subcore_barrier()
    pltpu.sync_copy(acc.at[pl.ds(rbase, ROWS_W)],
                    out_hbm.at[c, pl.ds(rbase, ROWS_W)])


_BLK = 2000


def _mm2p_body(x_ref, wa_ref, wb_ref, d0_ref, d1_ref, oa_ref, ob_ref):
    x = x_ref[...]
    dinv = lax.rsqrt(d0_ref[...] + d1_ref[...] + 1.0)
    oa_ref[...] = jnp.dot(x, wa_ref[...], preferred_element_type=jnp.float32) * dinv
    ob_ref[...] = jnp.dot(x, wb_ref[...], preferred_element_type=jnp.float32)


def _combine_body(p_ref, hs_ref, d0_ref, d1_ref, b_ref, w_ref, o_ref):
    dinv = lax.rsqrt(d0_ref[...] + d1_ref[...] + 1.0)
    z = dinv * (p_ref[0] + p_ref[1] + hs_ref[...]) + b_ref[...]
    z = jnp.maximum(z, 0.0)
    o_ref[...] = jnp.dot(z, w_ref[...], preferred_element_type=jnp.float32) * dinv


def _final_body(p_ref, hs_ref, d0_ref, d1_ref, b_ref, w_ref, xc_ref,
                bc_ref, o_ref):
    dinv = lax.rsqrt(d0_ref[...] + d1_ref[...] + 1.0)
    z = dinv * (p_ref[0] + p_ref[1] + hs_ref[...]) + b_ref[...]
    z = jnp.maximum(z, 0.0)
    o_ref[...] = (jnp.dot(z, w_ref[...], preferred_element_type=jnp.float32)
                  + xc_ref[...] + bc_ref[...])


def _row_spec(w):
    return pl.BlockSpec((_BLK, w), lambda i: (i, 0))


def _full_spec(r, w):
    return pl.BlockSpec((r, w), lambda i: (0, 0))


_part_spec = pl.BlockSpec((2, _BLK, D), lambda i: (0, i, 0))

_G = N // _BLK

_mm2p = pl.pallas_call(
    _mm2p_body,
    grid=(_G,),
    in_specs=[_row_spec(D), _full_spec(D, D), _full_spec(D, 64),
              _row_spec(1), _row_spec(1)],
    out_specs=[_row_spec(D), _row_spec(64)],
    out_shape=[jax.ShapeDtypeStruct((N, D), jnp.float32),
               jax.ShapeDtypeStruct((N, 64), jnp.float32)],
)

_combine = pl.pallas_call(
    _combine_body,
    grid=(_G,),
    in_specs=[_part_spec, _row_spec(D), _row_spec(1),
              _row_spec(1), _full_spec(1, D), _full_spec(D, D)],
    out_specs=_row_spec(D),
    out_shape=jax.ShapeDtypeStruct((N, D), jnp.float32),
)

_final = pl.pallas_call(
    _final_body,
    grid=(_G,),
    in_specs=[_part_spec, _row_spec(D), _row_spec(1),
              _row_spec(1), _full_spec(1, D), _full_spec(D, 64),
              _row_spec(64), _full_spec(1, 64)],
    out_specs=_row_spec(64),
    out_shape=jax.ShapeDtypeStruct((N, 64), jnp.float32),
)


def _pad_edges(src, dst):
    npad = E_WP - E_W
    w = jnp.arange(NW, dtype=jnp.int32).reshape(NW, 1)
    k = jnp.arange(npad, dtype=jnp.int32).reshape(1, npad)
    pad_src = (k * 41 + w * 13) % N
    pad_dst = N + (k + w * 7) % npad
    srcp = jnp.concatenate([src.reshape(NW, E_W), pad_src], axis=1)
    dstp = jnp.concatenate([dst.reshape(NW, E_W), pad_dst], axis=1)
    return srcp.reshape(NW, NCHP, CHP), dstp.reshape(NW, NCHP, CHP)


def kernel(x, edge_index, W1, b1, W2, b2, Wc, bc):
    src = edge_index[0].astype(jnp.int32)
    dst = edge_index[1].astype(jnp.int32)
    srcp, dstp = _pad_edges(src, dst)

    degp = _deg_partials(dstp)
    d0 = degp[0, 0, :N].reshape(N, 1)
    d1 = degp[1, 0, :N].reshape(N, 1)

    hs1, xc = _mm2p(x, W1, Wc[D:], d0, d1)
    p1 = _agg_partials(hs1, srcp, dstp)
    hs2 = _combine(p1, hs1, d0, d1,
                   b1.reshape(1, D), W2)
    p2 = _agg_partials(hs2, srcp, dstp)
    out = _final(p2, hs2, d0, d1,
                 b2.reshape(1, D), Wc[:D], xc,
                 bc.reshape(1, 64))
    return out


# speedup vs baseline: 1.2751x; 1.0651x over previous
import functools

import jax
import jax.numpy as jnp
from jax import lax
from jax.experimental import pallas as pl
from jax.experimental.pallas import tpu as pltpu
from jax.experimental.pallas import tpu_sc as plsc

N = 10000
D = 128
E = 320000
NC = 2
NS = 16
NW = NC * NS
E_W = E // NW
N_PAD = 10240
ROWS_W = N_PAD // NS
E_WP = 10240
CHP = 128
NCHP = E_WP // CHP
BLK_CH = 16
E_BLK = BLK_CH * CHP
NBLK = E_WP // E_BLK

_mesh = plsc.VectorSubcoreMesh(core_axis_name="c", subcore_axis_name="s")


@functools.partial(
    pl.kernel,
    out_type=jax.ShapeDtypeStruct((NC, 1, N_PAD), jnp.float32),
    mesh=_mesh,
    scratch_types=[
        pltpu.VMEM((NCHP, CHP), jnp.int32),
        pltpu.VMEM((CHP,), jnp.float32),
        pltpu.VMEM((ROWS_W,), jnp.float32),
        pltpu.VMEM_SHARED((N_PAD,), jnp.float32),
        pltpu.SemaphoreType.DMA,
    ],
)
def _deg_partials(dst_hbm, deg_out, didx, onesv, zb, acc, sem):
    c = lax.axis_index("c")
    s = lax.axis_index("s")
    wid = s * NC + c
    for i in range(CHP // 16):
        onesv[pl.ds(i * 16, 16)] = jnp.full((16,), 1.0, jnp.float32)
    for i in range(ROWS_W // 16):
        zb[pl.ds(i * 16, 16)] = jnp.zeros((16,), jnp.float32)
    pltpu.sync_copy(dst_hbm.at[wid], didx)
    pltpu.sync_copy(zb, acc.at[pl.ds(s * ROWS_W, ROWS_W)])
    plsc.subcore_barrier()

    def ebody(i, carry):
        pltpu.async_copy(onesv, acc.at[didx.at[i]], sem, add=True)
        return carry

    lax.fori_loop(0, NCHP, ebody, 0)

    def dbody(i, carry):
        pltpu.make_async_copy(onesv, acc.at[didx.at[0]], sem).wait()
        return carry

    lax.fori_loop(0, NCHP, dbody, 0)
    plsc.subcore_barrier()
    pltpu.sync_copy(acc.at[pl.ds(s * ROWS_W, ROWS_W)],
                    deg_out.at[c, 0, pl.ds(s * ROWS_W, ROWS_W)])


@functools.partial(
    pl.kernel,
    out_type=jax.ShapeDtypeStruct((NC, N_PAD, D), jnp.float32),
    mesh=_mesh,
    scratch_types=[
        pltpu.VMEM((BLK_CH, CHP), jnp.int32),
        pltpu.VMEM((BLK_CH, CHP), jnp.int32),
        pltpu.VMEM((BLK_CH, CHP), jnp.int32),
        pltpu.VMEM((BLK_CH, CHP), jnp.int32),
        pltpu.VMEM((CHP, D), jnp.float32),
        pltpu.VMEM((CHP, D), jnp.float32),
        pltpu.VMEM_SHARED((N_PAD, D), jnp.float32),
        pltpu.SemaphoreType.DMA,
        pltpu.SemaphoreType.DMA,
        pltpu.SemaphoreType.DMA,
        pltpu.SemaphoreType.DMA,
        pltpu.SemaphoreType.DMA,
    ],
)
def _agg_partials(hs_hbm, src_hbm, dst_hbm, out_hbm,
                  sidxA, sidxB, didxA, didxB, rows0, rows1, acc,
                  gsem0, gsem1, ssem0, ssem1, stsem):
    c = lax.axis_index("c")
    s = lax.axis_index("s")
    wid = s * NC + c
    rbase = s * ROWS_W
    rows = (rows0, rows1)
    gsem = (gsem0, gsem1)
    ssem = (ssem0, ssem1)
    sidxs = (sidxA, sidxB)
    didxs = (didxA, didxB)

    def stage_issue(m):
        ib = m % 2
        pltpu.async_copy(src_hbm.at[wid, pl.ds(m * BLK_CH, BLK_CH)],
                         sidxs[ib], stsem)
        pltpu.async_copy(dst_hbm.at[wid, pl.ds(m * BLK_CH, BLK_CH)],
                         didxs[ib], stsem)

    def stage_wait():
        for _ in range(2):
            pltpu.make_async_copy(src_hbm.at[wid, pl.ds(0, BLK_CH)],
                                  sidxA, stsem).wait()

    pltpu.sync_copy(src_hbm.at[wid, pl.ds(0, BLK_CH)], sidxA)
    pltpu.sync_copy(dst_hbm.at[wid, pl.ds(0, BLK_CH)], didxA)
    stage_issue(1)

    for i in range(32):
        for j in range(D // 16):
            rows1[i, pl.ds(j * 16, 16)] = jnp.zeros((16,), jnp.float32)

    def zbody(i, carry):
        pltpu.sync_copy(rows1.at[pl.ds(0, 32)], acc.at[pl.ds(rbase + i * 32, 32)])
        return carry

    lax.fori_loop(0, ROWS_W // 32, zbody, 0)
    plsc.subcore_barrier()

    descs = [None, None]
    for k in range(NCHP):
        b = k % 2
        blk = k // BLK_CH
        kk = k % BLK_CH
        ib = blk % 2
        if kk == 0 and blk >= 1:
            stage_wait()
        if kk == 2 and 1 <= blk <= NBLK - 2:
            stage_issue(blk + 1)
        if descs[b] is not None:
            descs[b].wait()
        pltpu.async_copy(hs_hbm.at[sidxs[ib].at[kk]], rows[b], gsem[b])
        if k >= 1:
            pb = (k - 1) % 2
            pib = ((k - 1) // BLK_CH) % 2
            pltpu.make_async_copy(hs_hbm.at[pl.ds(0, CHP)],
                                  rows[pb], gsem[pb]).wait()
            descs[pb] = pltpu.async_copy(
                rows[pb], acc.at[didxs[pib].at[(k - 1) % BLK_CH]],
                ssem[pb], add=True)
    lb = (NCHP - 1) % 2
    lib = ((NCHP - 1) // BLK_CH) % 2
    pltpu.make_async_copy(hs_hbm.at[pl.ds(0, CHP)], rows[lb], gsem[lb]).wait()
    dlast = pltpu.async_copy(rows[lb],
                             acc.at[didxs[lib].at[(NCHP - 1) % BLK_CH]],
                             ssem[lb], add=True)
    descs[1 - lb].wait()
    dlast.wait()
    plsc.subcore_barrier()
    pltpu.sync_copy(acc.at[pl.ds(rbase, ROWS_W)],
                    out_hbm.at[c, pl.ds(rbase, ROWS_W)])


_BLK = 2000


def _mm2p_body(x_ref, wa_ref, wb_ref, d0_ref, d1_ref, oa_ref, ob_ref):
    x = x_ref[...]
    dinv = lax.rsqrt(d0_ref[...] + d1_ref[...] + 1.0)
    oa_ref[...] = jnp.dot(x, wa_ref[...], preferred_element_type=jnp.float32) * dinv
    ob_ref[...] = jnp.dot(x, wb_ref[...], preferred_element_type=jnp.float32)


def _combine_body(p_ref, hs_ref, d0_ref, d1_ref, b_ref, w_ref, o_ref):
    dinv = lax.rsqrt(d0_ref[...] + d1_ref[...] + 1.0)
    z = dinv * (p_ref[0] + p_ref[1] + hs_ref[...]) + b_ref[...]
    z = jnp.maximum(z, 0.0)
    o_ref[...] = jnp.dot(z, w_ref[...], preferred_element_type=jnp.float32) * dinv


def _final_body(p_ref, hs_ref, d0_ref, d1_ref, b_ref, w_ref, xc_ref,
                bc_ref, o_ref):
    dinv = lax.rsqrt(d0_ref[...] + d1_ref[...] + 1.0)
    z = dinv * (p_ref[0] + p_ref[1] + hs_ref[...]) + b_ref[...]
    z = jnp.maximum(z, 0.0)
    o_ref[...] = (jnp.dot(z, w_ref[...], preferred_element_type=jnp.float32)
                  + xc_ref[...] + bc_ref[...])


def _row_spec(w):
    return pl.BlockSpec((_BLK, w), lambda i: (i, 0))


def _full_spec(r, w):
    return pl.BlockSpec((r, w), lambda i: (0, 0))


_part_spec = pl.BlockSpec((2, _BLK, D), lambda i: (0, i, 0))

_G = N // _BLK

_mm2p = pl.pallas_call(
    _mm2p_body,
    grid=(_G,),
    in_specs=[_row_spec(D), _full_spec(D, D), _full_spec(D, 64),
              _row_spec(1), _row_spec(1)],
    out_specs=[_row_spec(D), _row_spec(64)],
    out_shape=[jax.ShapeDtypeStruct((N, D), jnp.float32),
               jax.ShapeDtypeStruct((N, 64), jnp.float32)],
)

_combine = pl.pallas_call(
    _combine_body,
    grid=(_G,),
    in_specs=[_part_spec, _row_spec(D), _row_spec(1),
              _row_spec(1), _full_spec(1, D), _full_spec(D, D)],
    out_specs=_row_spec(D),
    out_shape=jax.ShapeDtypeStruct((N, D), jnp.float32),
)

_final = pl.pallas_call(
    _final_body,
    grid=(_G,),
    in_specs=[_part_spec, _row_spec(D), _row_spec(1),
              _row_spec(1), _full_spec(1, D), _full_spec(D, 64),
              _row_spec(64), _full_spec(1, 64)],
    out_specs=_row_spec(64),
    out_shape=jax.ShapeDtypeStruct((N, 64), jnp.float32),
)


def _pad_edges(src, dst):
    npad = E_WP - E_W
    w = jnp.arange(NW, dtype=jnp.int32).reshape(NW, 1)
    k = jnp.arange(npad, dtype=jnp.int32).reshape(1, npad)
    pad_src = (k * 41 + w * 13) % N
    pad_dst = N + (k + w * 7) % npad
    srcp = jnp.concatenate([src.reshape(NW, E_W), pad_src], axis=1)
    dstp = jnp.concatenate([dst.reshape(NW, E_W), pad_dst], axis=1)
    return srcp.reshape(NW, NCHP, CHP), dstp.reshape(NW, NCHP, CHP)


def kernel(x, edge_index, W1, b1, W2, b2, Wc, bc):
    src = edge_index[0].astype(jnp.int32)
    dst = edge_index[1].astype(jnp.int32)
    srcp, dstp = _pad_edges(src, dst)

    degp = _deg_partials(dstp)
    d0 = degp[0, 0, :N].reshape(N, 1)
    d1 = degp[1, 0, :N].reshape(N, 1)

    hs1, xc = _mm2p(x, W1, Wc[D:], d0, d1)
    p1 = _agg_partials(hs1, srcp, dstp)
    hs2 = _combine(p1, hs1, d0, d1,
                   b1.reshape(1, D), W2)
    p2 = _agg_partials(hs2, srcp, dstp)
    out = _final(p2, hs2, d0, d1,
                 b2.reshape(1, D), Wc[:D], xc,
                 bc.reshape(1, 64))
    return out


# speedup vs baseline: 1.2875x; 1.0098x over previous
import functools

import jax
import jax.numpy as jnp
from jax import lax
from jax.experimental import pallas as pl
from jax.experimental.pallas import tpu as pltpu
from jax.experimental.pallas import tpu_sc as plsc

N = 10000
D = 128
E = 320000
NC = 2
NS = 16
NW = NC * NS
E_W = E // NW
N_PAD = 10240
ROWS_W = N_PAD // NS
E_WP = 10240
CHP = 128
NCHP = E_WP // CHP
BLK_CH = 16
E_BLK = BLK_CH * CHP
NBLK = E_WP // E_BLK

_mesh = plsc.VectorSubcoreMesh(core_axis_name="c", subcore_axis_name="s")


@functools.partial(
    pl.kernel,
    out_type=jax.ShapeDtypeStruct((NC, 1, N_PAD), jnp.float32),
    mesh=_mesh,
    scratch_types=[
        pltpu.VMEM((NCHP, CHP), jnp.int32),
        pltpu.VMEM((CHP,), jnp.float32),
        pltpu.VMEM((ROWS_W,), jnp.float32),
        pltpu.VMEM_SHARED((N_PAD,), jnp.float32),
        pltpu.SemaphoreType.DMA,
    ],
)
def _deg_partials(dst_hbm, deg_out, didx, onesv, zb, acc, sem):
    c = lax.axis_index("c")
    s = lax.axis_index("s")
    wid = s * NC + c
    for i in range(CHP // 16):
        onesv[pl.ds(i * 16, 16)] = jnp.full((16,), 1.0, jnp.float32)
    for i in range(ROWS_W // 16):
        zb[pl.ds(i * 16, 16)] = jnp.zeros((16,), jnp.float32)
    pltpu.sync_copy(dst_hbm.at[wid], didx)
    pltpu.sync_copy(zb, acc.at[pl.ds(s * ROWS_W, ROWS_W)])
    plsc.subcore_barrier()

    def ebody(i, carry):
        pltpu.async_copy(onesv, acc.at[didx.at[i]], sem, add=True)
        return carry

    lax.fori_loop(0, NCHP, ebody, 0)

    def dbody(i, carry):
        pltpu.make_async_copy(onesv, acc.at[didx.at[0]], sem).wait()
        return carry

    lax.fori_loop(0, NCHP, dbody, 0)
    plsc.subcore_barrier()
    pltpu.sync_copy(acc.at[pl.ds(s * ROWS_W, ROWS_W)],
                    deg_out.at[c, 0, pl.ds(s * ROWS_W, ROWS_W)])


@functools.partial(
    pl.kernel,
    out_type=jax.ShapeDtypeStruct((NC, N_PAD, D), jnp.float32),
    mesh=_mesh,
    scratch_types=[
        pltpu.VMEM((BLK_CH, CHP), jnp.int32),
        pltpu.VMEM((BLK_CH, CHP), jnp.int32),
        pltpu.VMEM((BLK_CH, CHP), jnp.int32),
        pltpu.VMEM((BLK_CH, CHP), jnp.int32),
        pltpu.VMEM((CHP, D), jnp.float32),
        pltpu.VMEM((CHP, D), jnp.float32),
        pltpu.VMEM_SHARED((N_PAD, D), jnp.float32),
        pltpu.SemaphoreType.DMA,
        pltpu.SemaphoreType.DMA,
        pltpu.SemaphoreType.DMA,
        pltpu.SemaphoreType.DMA,
        pltpu.SemaphoreType.DMA,
    ],
)
def _agg_partials(hs_hbm, src_hbm, dst_hbm, out_hbm,
                  sidxA, sidxB, didxA, didxB, rows0, rows1, acc,
                  gsem0, gsem1, ssem0, ssem1, stsem):
    c = lax.axis_index("c")
    s = lax.axis_index("s")
    wid = s * NC + c
    rbase = s * ROWS_W
    rows = (rows0, rows1)
    gsem = (gsem0, gsem1)
    ssem = (ssem0, ssem1)
    sidxs = (sidxA, sidxB)
    didxs = (didxA, didxB)

    def stage_issue(m):
        ib = m % 2
        pltpu.async_copy(src_hbm.at[wid, pl.ds(m * BLK_CH, BLK_CH)],
                         sidxs[ib], stsem)
        pltpu.async_copy(dst_hbm.at[wid, pl.ds(m * BLK_CH, BLK_CH)],
                         didxs[ib], stsem)

    def stage_wait():
        for _ in range(2):
            pltpu.make_async_copy(src_hbm.at[wid, pl.ds(0, BLK_CH)],
                                  sidxA, stsem).wait()

    pltpu.sync_copy(src_hbm.at[wid, pl.ds(0, BLK_CH)], sidxA)
    pltpu.sync_copy(dst_hbm.at[wid, pl.ds(0, BLK_CH)], didxA)
    stage_issue(1)

    for i in range(32):
        for j in range(D // 16):
            rows1[i, pl.ds(j * 16, 16)] = jnp.zeros((16,), jnp.float32)

    def zbody(i, carry):
        pltpu.async_copy(rows1.at[pl.ds(0, 32)],
                         acc.at[pl.ds(rbase + i * 32, 32)], ssem0)
        return carry

    lax.fori_loop(0, ROWS_W // 32, zbody, 0)

    def zdrain(i, carry):
        pltpu.make_async_copy(rows1.at[pl.ds(0, 32)],
                              acc.at[pl.ds(rbase, 32)], ssem0).wait()
        return carry

    lax.fori_loop(0, ROWS_W // 32, zdrain, 0)
    plsc.subcore_barrier()

    descs = [None, None]
    for k in range(NCHP):
        b = k % 2
        blk = k // BLK_CH
        kk = k % BLK_CH
        ib = blk % 2
        if kk == 0 and blk >= 1:
            stage_wait()
        if kk == 2 and 1 <= blk <= NBLK - 2:
            stage_issue(blk + 1)
        if descs[b] is not None:
            descs[b].wait()
        pltpu.async_copy(hs_hbm.at[sidxs[ib].at[kk]], rows[b], gsem[b])
        if k >= 1:
            pb = (k - 1) % 2
            pib = ((k - 1) // BLK_CH) % 2
            pltpu.make_async_copy(hs_hbm.at[pl.ds(0, CHP)],
                                  rows[pb], gsem[pb]).wait()
            descs[pb] = pltpu.async_copy(
                rows[pb], acc.at[didxs[pib].at[(k - 1) % BLK_CH]],
                ssem[pb], add=True)
    lb = (NCHP - 1) % 2
    lib = ((NCHP - 1) // BLK_CH) % 2
    pltpu.make_async_copy(hs_hbm.at[pl.ds(0, CHP)], rows[lb], gsem[lb]).wait()
    dlast = pltpu.async_copy(rows[lb],
                             acc.at[didxs[lib].at[(NCHP - 1) % BLK_CH]],
                             ssem[lb], add=True)
    descs[1 - lb].wait()
    dlast.wait()
    plsc.subcore_barrier()
    pltpu.sync_copy(acc.at[pl.ds(rbase, ROWS_W)],
                    out_hbm.at[c, pl.ds(rbase, ROWS_W)])


_BLK = 2000


def _mm2p_body(x_ref, wa_ref, wb_ref, d0_ref, d1_ref, oa_ref, ob_ref):
    x = x_ref[...]
    dinv = lax.rsqrt(d0_ref[...] + d1_ref[...] + 1.0)
    oa_ref[...] = jnp.dot(x, wa_ref[...], preferred_element_type=jnp.float32) * dinv
    ob_ref[...] = jnp.dot(x, wb_ref[...], preferred_element_type=jnp.float32)


def _combine_body(p_ref, hs_ref, d0_ref, d1_ref, b_ref, w_ref, o_ref):
    dinv = lax.rsqrt(d0_ref[...] + d1_ref[...] + 1.0)
    z = dinv * (p_ref[0] + p_ref[1] + hs_ref[...]) + b_ref[...]
    z = jnp.maximum(z, 0.0)
    o_ref[...] = jnp.dot(z, w_ref[...], preferred_element_type=jnp.float32) * dinv


def _final_body(p_ref, hs_ref, d0_ref, d1_ref, b_ref, w_ref, xc_ref,
                bc_ref, o_ref):
    dinv = lax.rsqrt(d0_ref[...] + d1_ref[...] + 1.0)
    z = dinv * (p_ref[0] + p_ref[1] + hs_ref[...]) + b_ref[...]
    z = jnp.maximum(z, 0.0)
    o_ref[...] = (jnp.dot(z, w_ref[...], preferred_element_type=jnp.float32)
                  + xc_ref[...] + bc_ref[...])


def _row_spec(w):
    return pl.BlockSpec((_BLK, w), lambda i: (i, 0))


def _full_spec(r, w):
    return pl.BlockSpec((r, w), lambda i: (0, 0))


_part_spec = pl.BlockSpec((2, _BLK, D), lambda i: (0, i, 0))

_G = N // _BLK

_mm2p = pl.pallas_call(
    _mm2p_body,
    grid=(_G,),
    in_specs=[_row_spec(D), _full_spec(D, D), _full_spec(D, 64),
              _row_spec(1), _row_spec(1)],
    out_specs=[_row_spec(D), _row_spec(64)],
    out_shape=[jax.ShapeDtypeStruct((N, D), jnp.float32),
               jax.ShapeDtypeStruct((N, 64), jnp.float32)],
)

_combine = pl.pallas_call(
    _combine_body,
    grid=(_G,),
    in_specs=[_part_spec, _row_spec(D), _row_spec(1),
              _row_spec(1), _full_spec(1, D), _full_spec(D, D)],
    out_specs=_row_spec(D),
    out_shape=jax.ShapeDtypeStruct((N, D), jnp.float32),
)

_final = pl.pallas_call(
    _final_body,
    grid=(_G,),
    in_specs=[_part_spec, _row_spec(D), _row_spec(1),
              _row_spec(1), _full_spec(1, D), _full_spec(D, 64),
              _row_spec(64), _full_spec(1, 64)],
    out_specs=_row_spec(64),
    out_shape=jax.ShapeDtypeStruct((N, 64), jnp.float32),
)


def _pad_edges(src, dst):
    npad = E_WP - E_W
    w = jnp.arange(NW, dtype=jnp.int32).reshape(NW, 1)
    k = jnp.arange(npad, dtype=jnp.int32).reshape(1, npad)
    pad_src = (k * 41 + w * 13) % N
    pad_dst = N + (k + w * 7) % npad
    srcp = jnp.concatenate([src.reshape(NW, E_W), pad_src], axis=1)
    dstp = jnp.concatenate([dst.reshape(NW, E_W), pad_dst], axis=1)
    return srcp.reshape(NW, NCHP, CHP), dstp.reshape(NW, NCHP, CHP)


def kernel(x, edge_index, W1, b1, W2, b2, Wc, bc):
    src = edge_index[0].astype(jnp.int32)
    dst = edge_index[1].astype(jnp.int32)
    srcp, dstp = _pad_edges(src, dst)

    degp = _deg_partials(dstp)
    d0 = degp[0, 0, :N].reshape(N, 1)
    d1 = degp[1, 0, :N].reshape(N, 1)

    hs1, xc = _mm2p(x, W1, Wc[D:], d0, d1)
    p1 = _agg_partials(hs1, srcp, dstp)
    hs2 = _combine(p1, hs1, d0, d1,
                   b1.reshape(1, D), W2)
    p2 = _agg_partials(hs2, srcp, dstp)
    out = _final(p2, hs2, d0, d1,
                 b2.reshape(1, D), Wc[:D], xc,
                 bc.reshape(1, 64))
    return out
